# baseline (speedup 1.0000x reference)
import functools

import jax
import jax.numpy as jnp
from jax import lax
from jax.experimental import pallas as pl
from jax.experimental.pallas import tpu as pltpu
from jax.experimental.pallas import tpu_sc as plsc

V = 100000
D = 300
DPK = 256
DW = 384
R = 10
N_RELS = 40
B = 4096

NC, NS = 2, 16
NW = NC * NS
HB = B // 2
SIDE_PER_W = HB // NW

BLK = 1024
NBLK = HB // BLK

PAD_RB = 8192
NPB = -(-V // PAD_RB)


def _pad_body(xt_ref, o_ref):
    xt = xt_ref[...].T.astype(jnp.bfloat16)
    z = lax.bitcast_convert_type(xt, jnp.uint16).astype(jnp.uint32)
    zw = jnp.pad(z, ((0, 0), (0, 2 * DPK - D)))
    hi = pltpu.roll(zw, DPK, 1)[:, :DPK]
    packed = zw[:, :DPK] | (hi << 16)
    o_ref[...] = lax.bitcast_convert_type(packed, jnp.int32)


def _pad_table(term_table_t):
    return pl.pallas_call(
        _pad_body,
        grid=(NPB,),
        in_specs=[pl.BlockSpec((D, PAD_RB), lambda i: (0, i))],
        out_specs=pl.BlockSpec((PAD_RB, DPK), lambda i: (i, 0)),
        out_shape=jax.ShapeDtypeStruct((V, DPK), jnp.int32),
        compiler_params=pltpu.CompilerParams(
            dimension_semantics=("parallel",)),
    )(term_table_t)


def _sc_gather(table, t_l, t_r):
    mesh = plsc.VectorSubcoreMesh(core_axis_name="c", subcore_axis_name="s")

    @functools.partial(
        pl.kernel,
        mesh=mesh,
        out_type=(
            jax.ShapeDtypeStruct((HB, DPK), jnp.int32),
            jax.ShapeDtypeStruct((HB, DPK), jnp.int32),
        ),
        scratch_types=[
            pltpu.VMEM((SIDE_PER_W,), jnp.int32),
            pltpu.VMEM((SIDE_PER_W, DPK), jnp.int32),
            pltpu.SemaphoreType.DMA,
        ],
    )
    def gather_kernel(table_hbm, tl_hbm, tr_hbm, ol_hbm, or_hbm,
                      idx_v, rows_v, sem):
        wid = lax.axis_index("s") * NC + lax.axis_index("c")
        base = wid * SIDE_PER_W
        for i_hbm, o_hbm in ((tl_hbm, ol_hbm), (tr_hbm, or_hbm)):
            pltpu.sync_copy(i_hbm.at[pl.ds(base, SIDE_PER_W)], idx_v)
            pltpu.async_copy(table_hbm.at[idx_v], rows_v, sem).wait()
            pltpu.sync_copy(rows_v, o_hbm.at[pl.ds(base, SIDE_PER_W)])

    return gather_kernel(table, t_l, t_r)


def _unpack(v):
    lo = lax.bitcast_convert_type(v << 16, jnp.float32)
    hi = lax.bitcast_convert_type(
        v & jnp.int32(-65536), jnp.float32)
    return jnp.concatenate([lo, hi[:, :DW - DPK]], axis=1)


def _tc_body(l_ref, r_ref, rel1h_ref, relt_ref, w_ref, bb_ref, tm_ref,
             to_ref, out_ref):
    rel_emb = jnp.dot(rel1h_ref[...], relt_ref[...],
                      preferred_element_type=jnp.float32)
    l32 = _unpack(l_ref[...])
    r_sel = _unpack(r_ref[...])
    p = jnp.zeros((BLK, DW), jnp.float32)
    w = w_ref[...]
    for k in range(R):
        a = (l32 * rel_emb[:, k:k + 1]).astype(jnp.bfloat16)
        p = p + jnp.dot(a, w[k], preferred_element_type=jnp.float32)
    energy = (jnp.sum(p * r_sel, axis=1, keepdims=True)
              + jnp.sum(rel_emb * bb_ref[...], axis=1, keepdims=True))
    out_ref[...] = energy * tm_ref[0, 0] + to_ref[0, 0]


def kernel(rels, terms_L, terms_R, term_table, rel_table, bil_w, bil_b,
           truth_multiplier, truth_offset):
    tpad = _pad_table(term_table.T)

    rel1h = (rels[:, None] == jnp.arange(N_RELS, dtype=jnp.int32)[None, :])
    rel1h = rel1h.astype(jnp.float32)
    w_bf = jnp.pad(bil_w.astype(jnp.bfloat16),
                   ((0, 0), (0, DW - D), (0, DW - D)))
    bb = bil_b.reshape(1, R)
    tm = truth_multiplier.reshape(1, 1)
    to = truth_offset.reshape(1, 1)

    def compute(gl, gr, r1h):
        return pl.pallas_call(
            _tc_body,
            grid=(NBLK,),
            in_specs=[
                pl.BlockSpec((BLK, DPK), lambda i: (i, 0)),
                pl.BlockSpec((BLK, DPK), lambda i: (i, 0)),
                pl.BlockSpec((BLK, N_RELS), lambda i: (i, 0)),
                pl.BlockSpec((N_RELS, R), lambda i: (0, 0)),
                pl.BlockSpec((R, DW, DW), lambda i: (0, 0, 0)),
                pl.BlockSpec((1, R), lambda i: (0, 0)),
                pl.BlockSpec((1, 1), lambda i: (0, 0)),
                pl.BlockSpec((1, 1), lambda i: (0, 0)),
            ],
            out_specs=pl.BlockSpec((BLK, 1), lambda i: (i, 0)),
            out_shape=jax.ShapeDtypeStruct((HB, 1), jnp.float32),
            compiler_params=pltpu.CompilerParams(
                dimension_semantics=("parallel",)),
        )(gl, gr, r1h, rel_table, w_bf, bb, tm, to)

    gl1, gr1 = _sc_gather(tpad, terms_L[:HB], terms_R[:HB])
    gl2, gr2 = _sc_gather(tpad, terms_L[HB:], terms_R[HB:])
    out1 = compute(gl1, gr1, rel1h[:HB])
    out2 = compute(gl2, gr2, rel1h[HB:])
    return jnp.concatenate([out1, out2], axis=0).reshape(B)


# speedup vs baseline: 1.0207x; 1.0207x over previous
import functools

import jax
import jax.numpy as jnp
from jax import lax
from jax.experimental import pallas as pl
from jax.experimental.pallas import tpu as pltpu
from jax.experimental.pallas import tpu_sc as plsc

V = 100000
D = 300
DPK = 256
DW = 384
R = 10
N_RELS = 40
B = 4096

NC, NS = 2, 16
NW = NC * NS
HB = B
SIDE_PER_W = HB // NW

BLK = 1024
NBLK = HB // BLK

PAD_RB = 8192
NPB = -(-V // PAD_RB)


def _pad_body(xt_ref, o_ref):
    xt = xt_ref[...].T.astype(jnp.bfloat16)
    z = lax.bitcast_convert_type(xt, jnp.uint16).astype(jnp.uint32)
    zw = jnp.pad(z, ((0, 0), (0, 2 * DPK - D)))
    hi = pltpu.roll(zw, DPK, 1)[:, :DPK]
    packed = zw[:, :DPK] | (hi << 16)
    o_ref[...] = lax.bitcast_convert_type(packed, jnp.int32)


def _pad_table(term_table_t):
    return pl.pallas_call(
        _pad_body,
        grid=(NPB,),
        in_specs=[pl.BlockSpec((D, PAD_RB), lambda i: (0, i))],
        out_specs=pl.BlockSpec((PAD_RB, DPK), lambda i: (i, 0)),
        out_shape=jax.ShapeDtypeStruct((V, DPK), jnp.int32),
        compiler_params=pltpu.CompilerParams(
            dimension_semantics=("parallel",)),
    )(term_table_t)


def _sc_gather(table, t_l, t_r):
    mesh = plsc.VectorSubcoreMesh(core_axis_name="c", subcore_axis_name="s")

    @functools.partial(
        pl.kernel,
        mesh=mesh,
        out_type=(
            jax.ShapeDtypeStruct((HB, DPK), jnp.int32),
            jax.ShapeDtypeStruct((HB, DPK), jnp.int32),
        ),
        scratch_types=[
            pltpu.VMEM((SIDE_PER_W,), jnp.int32),
            pltpu.VMEM((SIDE_PER_W,), jnp.int32),
            pltpu.VMEM((SIDE_PER_W, DPK), jnp.int32),
            pltpu.VMEM((SIDE_PER_W, DPK), jnp.int32),
            pltpu.SemaphoreType.DMA,
            pltpu.SemaphoreType.DMA,
        ],
    )
    def gather_kernel(table_hbm, tl_hbm, tr_hbm, ol_hbm, or_hbm,
                      idx_l, idx_r, rows_l, rows_r, sem_l, sem_r):
        wid = lax.axis_index("s") * NC + lax.axis_index("c")
        base = wid * SIDE_PER_W
        pltpu.sync_copy(tl_hbm.at[pl.ds(base, SIDE_PER_W)], idx_l)
        pltpu.sync_copy(tr_hbm.at[pl.ds(base, SIDE_PER_W)], idx_r)
        cp_l = pltpu.async_copy(table_hbm.at[idx_l], rows_l, sem_l)
        cp_r = pltpu.async_copy(table_hbm.at[idx_r], rows_r, sem_r)
        cp_l.wait()
        pltpu.sync_copy(rows_l, ol_hbm.at[pl.ds(base, SIDE_PER_W)])
        cp_r.wait()
        pltpu.sync_copy(rows_r, or_hbm.at[pl.ds(base, SIDE_PER_W)])

    return gather_kernel(table, t_l, t_r)


def _unpack(v):
    lo = lax.bitcast_convert_type(v << 16, jnp.float32)
    hi = lax.bitcast_convert_type(
        v & jnp.int32(-65536), jnp.float32)
    return jnp.concatenate([lo, hi[:, :DW - DPK]], axis=1)


def _tc_body(l_ref, r_ref, rel1h_ref, relt_ref, w_ref, bb_ref, tm_ref,
             to_ref, out_ref):
    rel_emb = jnp.dot(rel1h_ref[...], relt_ref[...],
                      preferred_element_type=jnp.float32)
    l32 = _unpack(l_ref[...])
    r_sel = _unpack(r_ref[...])
    p = jnp.zeros((BLK, DW), jnp.float32)
    w = w_ref[...]
    for k in range(R):
        a = (l32 * rel_emb[:, k:k + 1]).astype(jnp.bfloat16)
        p = p + jnp.dot(a, w[k], preferred_element_type=jnp.float32)
    energy = (jnp.sum(p * r_sel, axis=1, keepdims=True)
              + jnp.sum(rel_emb * bb_ref[...], axis=1, keepdims=True))
    out_ref[...] = energy * tm_ref[0, 0] + to_ref[0, 0]


def kernel(rels, terms_L, terms_R, term_table, rel_table, bil_w, bil_b,
           truth_multiplier, truth_offset):
    tpad = _pad_table(term_table.T)

    rel1h = (rels[:, None] == jnp.arange(N_RELS, dtype=jnp.int32)[None, :])
    rel1h = rel1h.astype(jnp.float32)
    w_bf = jnp.pad(bil_w.astype(jnp.bfloat16),
                   ((0, 0), (0, DW - D), (0, DW - D)))
    bb = bil_b.reshape(1, R)
    tm = truth_multiplier.reshape(1, 1)
    to = truth_offset.reshape(1, 1)

    def compute(gl, gr, r1h):
        return pl.pallas_call(
            _tc_body,
            grid=(NBLK,),
            in_specs=[
                pl.BlockSpec((BLK, DPK), lambda i: (i, 0)),
                pl.BlockSpec((BLK, DPK), lambda i: (i, 0)),
                pl.BlockSpec((BLK, N_RELS), lambda i: (i, 0)),
                pl.BlockSpec((N_RELS, R), lambda i: (0, 0)),
                pl.BlockSpec((R, DW, DW), lambda i: (0, 0, 0)),
                pl.BlockSpec((1, R), lambda i: (0, 0)),
                pl.BlockSpec((1, 1), lambda i: (0, 0)),
                pl.BlockSpec((1, 1), lambda i: (0, 0)),
            ],
            out_specs=pl.BlockSpec((BLK, 1), lambda i: (i, 0)),
            out_shape=jax.ShapeDtypeStruct((HB, 1), jnp.float32),
            compiler_params=pltpu.CompilerParams(
                dimension_semantics=("parallel",)),
        )(gl, gr, r1h, rel_table, w_bf, bb, tm, to)

    gl, gr = _sc_gather(tpad, terms_L, terms_R)
    return compute(gl, gr, rel1h).reshape(B)


# speedup vs baseline: 1.0233x; 1.0025x over previous
import functools

import jax
import jax.numpy as jnp
from jax import lax
from jax.experimental import pallas as pl
from jax.experimental.pallas import tpu as pltpu
from jax.experimental.pallas import tpu_sc as plsc

V = 100000
D = 300
DPK = 256
DW = 384
R = 10
N_RELS = 40
B = 4096

NC, NS = 2, 16
NW = NC * NS
HB = B
SIDE_PER_W = HB // NW

BLK = 1024
NBLK = HB // BLK

PAD_RB = 10240
NPB = -(-V // PAD_RB)


def _pad_body(xt_ref, o_ref):
    xt = xt_ref[...].T.astype(jnp.bfloat16)
    z = lax.bitcast_convert_type(xt, jnp.uint16).astype(jnp.uint32)
    zw = jnp.pad(z, ((0, 0), (0, 2 * DPK - D)))
    hi = pltpu.roll(zw, DPK, 1)[:, :DPK]
    packed = zw[:, :DPK] | (hi << 16)
    o_ref[...] = lax.bitcast_convert_type(packed, jnp.int32)


def _pad_table(term_table_t):
    return pl.pallas_call(
        _pad_body,
        grid=(NPB,),
        in_specs=[pl.BlockSpec((D, PAD_RB), lambda i: (0, i))],
        out_specs=pl.BlockSpec((PAD_RB, DPK), lambda i: (i, 0)),
        out_shape=jax.ShapeDtypeStruct((V, DPK), jnp.int32),
        compiler_params=pltpu.CompilerParams(
            dimension_semantics=("parallel",)),
    )(term_table_t)


def _sc_gather(table, t_l, t_r):
    mesh = plsc.VectorSubcoreMesh(core_axis_name="c", subcore_axis_name="s")

    @functools.partial(
        pl.kernel,
        mesh=mesh,
        out_type=(
            jax.ShapeDtypeStruct((HB, DPK), jnp.int32),
            jax.ShapeDtypeStruct((HB, DPK), jnp.int32),
        ),
        scratch_types=[
            pltpu.VMEM((SIDE_PER_W,), jnp.int32),
            pltpu.VMEM((SIDE_PER_W,), jnp.int32),
            pltpu.VMEM((SIDE_PER_W, DPK), jnp.int32),
            pltpu.VMEM((SIDE_PER_W, DPK), jnp.int32),
            pltpu.SemaphoreType.DMA,
            pltpu.SemaphoreType.DMA,
        ],
    )
    def gather_kernel(table_hbm, tl_hbm, tr_hbm, ol_hbm, or_hbm,
                      idx_l, idx_r, rows_l, rows_r, sem_l, sem_r):
        wid = lax.axis_index("s") * NC + lax.axis_index("c")
        base = wid * SIDE_PER_W
        pltpu.sync_copy(tl_hbm.at[pl.ds(base, SIDE_PER_W)], idx_l)
        pltpu.sync_copy(tr_hbm.at[pl.ds(base, SIDE_PER_W)], idx_r)
        cp_l = pltpu.async_copy(table_hbm.at[idx_l], rows_l, sem_l)
        cp_r = pltpu.async_copy(table_hbm.at[idx_r], rows_r, sem_r)
        cp_l.wait()
        pltpu.sync_copy(rows_l, ol_hbm.at[pl.ds(base, SIDE_PER_W)])
        cp_r.wait()
        pltpu.sync_copy(rows_r, or_hbm.at[pl.ds(base, SIDE_PER_W)])

    return gather_kernel(table, t_l, t_r)


def _unpack(v):
    lo = lax.bitcast_convert_type(v << 16, jnp.float32)
    hi = lax.bitcast_convert_type(
        v & jnp.int32(-65536), jnp.float32)
    return jnp.concatenate([lo, hi[:, :DW - DPK]], axis=1)


def _tc_body(l_ref, r_ref, rel1h_ref, relt_ref, w_ref, bb_ref, tm_ref,
             to_ref, out_ref):
    rel_emb = jnp.dot(rel1h_ref[...], relt_ref[...],
                      preferred_element_type=jnp.float32)
    l32 = _unpack(l_ref[...])
    r_sel = _unpack(r_ref[...])
    p = jnp.zeros((BLK, DW), jnp.float32)
    w = w_ref[...]
    for k in range(R):
        a = (l32 * rel_emb[:, k:k + 1]).astype(jnp.bfloat16)
        p = p + jnp.dot(a, w[k], preferred_element_type=jnp.float32)
    energy = (jnp.sum(p * r_sel, axis=1, keepdims=True)
              + jnp.sum(rel_emb * bb_ref[...], axis=1, keepdims=True))
    out_ref[...] = energy * tm_ref[0, 0] + to_ref[0, 0]


def kernel(rels, terms_L, terms_R, term_table, rel_table, bil_w, bil_b,
           truth_multiplier, truth_offset):
    tpad = _pad_table(term_table.T)

    rel1h = (rels[:, None] == jnp.arange(N_RELS, dtype=jnp.int32)[None, :])
    rel1h = rel1h.astype(jnp.float32)
    w_bf = jnp.pad(bil_w.astype(jnp.bfloat16),
                   ((0, 0), (0, DW - D), (0, DW - D)))
    bb = bil_b.reshape(1, R)
    tm = truth_multiplier.reshape(1, 1)
    to = truth_offset.reshape(1, 1)

    def compute(gl, gr, r1h):
        return pl.pallas_call(
            _tc_body,
            grid=(NBLK,),
            in_specs=[
                pl.BlockSpec((BLK, DPK), lambda i: (i, 0)),
                pl.BlockSpec((BLK, DPK), lambda i: (i, 0)),
                pl.BlockSpec((BLK, N_RELS), lambda i: (i, 0)),
                pl.BlockSpec((N_RELS, R), lambda i: (0, 0)),
                pl.BlockSpec((R, DW, DW), lambda i: (0, 0, 0)),
                pl.BlockSpec((1, R), lambda i: (0, 0)),
                pl.BlockSpec((1, 1), lambda i: (0, 0)),
                pl.BlockSpec((1, 1), lambda i: (0, 0)),
            ],
            out_specs=pl.BlockSpec((BLK, 1), lambda i: (i, 0)),
            out_shape=jax.ShapeDtypeStruct((HB, 1), jnp.float32),
            compiler_params=pltpu.CompilerParams(
                dimension_semantics=("parallel",)),
        )(gl, gr, r1h, rel_table, w_bf, bb, tm, to)

    gl, gr = _sc_gather(tpad, terms_L, terms_R)
    return compute(gl, gr, rel1h).reshape(B)
